# baseline (speedup 1.0000x reference)
import jax
import jax.numpy as jnp
from jax import lax
from jax.experimental import pallas as pl
from jax.experimental.pallas import tpu as pltpu
from jax.experimental.pallas import tpu_sc as plsc

_B, _C, _H, _W = 32, 384, 28, 28
_HW = _H * _W
_K = 4
_CG = _C // _K
_S = 24
_NSL = _CG // _S
_NV = _S * _HW // 16


def _sc_body(x_hbm, out_hbm, b0, b1, b2, b3, sem):
    cid = lax.axis_index("c")
    sid = lax.axis_index("s")
    wid = sid * 2 + cid
    bufs = (b0, b1, b2, b3)

    for s in range(_NSL):
        chs = [j * _CG + s * _S for j in range(_K)]
        cps = [
            pltpu.make_async_copy(x_hbm.at[wid, pl.ds(chs[j], _S)], bufs[j], sem)
            for j in range(_K)
        ]
        for cp in cps:
            cp.start()
        for cp in cps:
            cp.wait()

        def body(i, carry):
            c0 = i // _H
            h = i % _H
            for w0 in (0, _W - 16):
                sl = pl.ds(w0, 16)
                a = b0[c0, h, sl]
                b = b1[c0, h, sl]
                cc = b2[c0, h, sl]
                d = b3[c0, h, sl]
                lo1 = jnp.minimum(a, b)
                hi1 = jnp.maximum(a, b)
                lo2 = jnp.minimum(cc, d)
                hi2 = jnp.maximum(cc, d)
                b0[c0, h, sl] = jnp.maximum(hi1, hi2)
                b3[c0, h, sl] = jnp.minimum(lo1, lo2)
                m1 = jnp.minimum(hi1, hi2)
                m2 = jnp.maximum(lo1, lo2)
                b1[c0, h, sl] = jnp.maximum(m1, m2)
                b2[c0, h, sl] = jnp.minimum(m1, m2)
            return carry

        lax.fori_loop(0, _S * _H, body, 0)

        ocps = [
            pltpu.make_async_copy(bufs[j], out_hbm.at[wid, pl.ds(chs[j], _S)], sem)
            for j in range(_K)
        ]
        for cp in ocps:
            cp.start()
        for cp in ocps:
            cp.wait()


def kernel(x):
    mesh = plsc.VectorSubcoreMesh(core_axis_name="c", subcore_axis_name="s")
    kfn = pl.kernel(
        _sc_body,
        mesh=mesh,
        out_type=jax.ShapeDtypeStruct((_B, _C, _H, _W), jnp.float32),
        scratch_types=[pltpu.VMEM((_S, _H, _W), jnp.float32) for _ in range(_K)]
        + [pltpu.SemaphoreType.DMA],
        compiler_params=pltpu.CompilerParams(use_tc_tiling_on_sc=False),
    )
    return kfn(x)


# speedup vs baseline: 1.1763x; 1.1763x over previous
import jax
import jax.numpy as jnp
from jax import lax
from jax.experimental import pallas as pl
from jax.experimental.pallas import tpu as pltpu
from jax.experimental.pallas import tpu_sc as plsc

_B, _C, _H, _W = 32, 384, 28, 28
_HW = _H * _W
_K = 4
_CG = _C // _K
_S = 24
_NSL = _CG // _S

_B_SC = 16
_B_TC = _B - _B_SC
_NSUB = 32 // _B_SC
_SPS = _NSL // _NSUB

_CH = _S * _HW
_NV = _CH // 16


def _sort4(a, b, c, d):
    lo1 = jnp.minimum(a, b)
    hi1 = jnp.maximum(a, b)
    lo2 = jnp.minimum(c, d)
    hi2 = jnp.maximum(c, d)
    m1 = jnp.minimum(hi1, hi2)
    m2 = jnp.maximum(lo1, lo2)
    return (
        jnp.maximum(hi1, hi2),
        jnp.maximum(m1, m2),
        jnp.minimum(m1, m2),
        jnp.minimum(lo1, lo2),
    )


def _sc_body(x_hbm, out_hbm, b0, b1, b2, b3, sem):
    cid = lax.axis_index("c")
    sid = lax.axis_index("s")
    wid = sid * 2 + cid
    batch = wid // _NSUB
    s0 = (wid % _NSUB) * _SPS
    base = batch * (_C * _HW)
    bufs = (b0, b1, b2, b3)

    for si in range(_SPS):
        offs = [base + (j * _CG + (s0 + si) * _S) * _HW for j in range(_K)]
        cps = [
            pltpu.make_async_copy(x_hbm.at[pl.ds(offs[j], _CH)], bufs[j], sem)
            for j in range(_K)
        ]
        for cp in cps:
            cp.start()
        for cp in cps:
            cp.wait()

        def body(i, carry):
            sl = pl.ds(i * 16, 16)
            r0, r1, r2, r3 = _sort4(b0[sl], b1[sl], b2[sl], b3[sl])
            b0[sl] = r0
            b1[sl] = r1
            b2[sl] = r2
            b3[sl] = r3
            return carry

        lax.fori_loop(0, _NV, body, 0)

        ocps = [
            pltpu.make_async_copy(bufs[j], out_hbm.at[pl.ds(offs[j], _CH)], sem)
            for j in range(_K)
        ]
        for cp in ocps:
            cp.start()
        for cp in ocps:
            cp.wait()


def _sc_sort(x_sc_flat):
    mesh = plsc.VectorSubcoreMesh(core_axis_name="c", subcore_axis_name="s")
    kfn = pl.kernel(
        _sc_body,
        mesh=mesh,
        out_type=jax.ShapeDtypeStruct((_B_SC * _C * _HW,), jnp.float32),
        scratch_types=[pltpu.VMEM((_CH,), jnp.float32) for _ in range(_K)]
        + [pltpu.SemaphoreType.DMA],
    )
    return kfn(x_sc_flat)


def _tc_body(x_ref, o_ref):
    a = x_ref[0, 0 * _CG:1 * _CG]
    b = x_ref[0, 1 * _CG:2 * _CG]
    c = x_ref[0, 2 * _CG:3 * _CG]
    d = x_ref[0, 3 * _CG:4 * _CG]
    r0, r1, r2, r3 = _sort4(a, b, c, d)
    o_ref[0, 0 * _CG:1 * _CG] = r0
    o_ref[0, 1 * _CG:2 * _CG] = r1
    o_ref[0, 2 * _CG:3 * _CG] = r2
    o_ref[0, 3 * _CG:4 * _CG] = r3


def _tc_sort(x):
    return pl.pallas_call(
        _tc_body,
        grid=(_B_TC,),
        in_specs=[
            pl.BlockSpec((1, _C, _H, _W), lambda b: (b, 0, 0, 0)),
        ],
        out_specs=pl.BlockSpec((1, _C, _H, _W), lambda b: (b, 0, 0, 0)),
        out_shape=jax.ShapeDtypeStruct((_B_TC, _C, _H, _W), jnp.float32),
    )(x)


def kernel(x):
    out_tc = _tc_sort(x)
    out_sc = _sc_sort(x[_B_TC:].reshape(-1))
    return jnp.concatenate(
        [out_tc, out_sc.reshape(_B_SC, _C, _H, _W)], axis=0
    )
